# baseline (speedup 1.0000x reference)
import jax
import jax.numpy as jnp
from jax import lax
from jax.experimental import pallas as pl
from jax.experimental.pallas import tpu as pltpu
from jax.experimental.pallas import tpu_sc as plsc

DIM = 64
EPS = 1e-5
NC, NS = 2, 16
NW = NC * NS
RPC = 4
NQ = DIM // 16
LANES = 128
PADF = 32


def _rsqrt_newton(x):
    i = lax.bitcast_convert_type(x, jnp.int32)
    i = jnp.int32(0x5F3759DF) - (i >> 1)
    y = lax.bitcast_convert_type(i, jnp.float32)
    for _ in range(3):
        y = y * (1.5 - 0.5 * x * y * y)
    return y


def _concat_tables(t0, t1):
    v = t0.shape[0]
    blk = 2000
    assert v % blk == 0

    def body(a_ref, b_ref, o_ref):
        o_ref[...] = jnp.concatenate([a_ref[...], b_ref[...]], axis=-1)

    return pl.pallas_call(
        body,
        grid=(v // blk,),
        in_specs=[
            pl.BlockSpec((blk, DIM), lambda i: (i, 0)),
            pl.BlockSpec((blk, DIM), lambda i: (i, 0)),
        ],
        out_specs=pl.BlockSpec((blk, 2 * DIM), lambda i: (i, 0)),
        out_shape=jax.ShapeDtypeStruct((v, 2 * DIM), jnp.float32),
    )(t0, t1)


def _fused_sc(i0, i1, ct, gamma, beta, fields):
    batch = i0.shape[0]
    rows_w = batch // NW
    chunks = rows_w // RPC
    lpc = RPC * fields
    assert chunks % 2 == 0 and lpc % 8 == 0 and lpc <= LANES
    groups = (lpc + 15) // 16
    mesh = plsc.VectorSubcoreMesh(core_axis_name="c", subcore_axis_name="s")

    def body(i0_hbm, i1_hbm, ct_hbm, g_hbm, b_hbm, out_hbm,
             ir0, ir1, if0, if1, gb_v,
             r0a, r1a, r0b, r1b, oa, ob,
             sga, sgb, soa, sob):
        wid = lax.axis_index("s") * NC + lax.axis_index("c")
        row0 = wid * rows_w

        pltpu.sync_copy(i0_hbm.at[pl.ds(row0, rows_w)], ir0)
        pltpu.sync_copy(i1_hbm.at[pl.ds(row0, rows_w)], ir1)
        pltpu.sync_copy(g_hbm, gb_v.at[0])
        pltpu.sync_copy(b_hbm, gb_v.at[1])
        gv = [gb_v[0, pl.ds(16 * q, 16)] for q in range(NQ)]
        bv = [gb_v[1, pl.ds(16 * q, 16)] for q in range(NQ)]

        rowc = []
        colc = []
        for g in range(groups):
            t = jnp.arange(16, dtype=jnp.int32) + 16 * g
            rowc.append(t // fields)
            colc.append(t % fields)

        def flatten(j, carry):
            base = j * RPC
            for g in range(groups):
                sl = pl.ds(16 * g, 16)
                row = jnp.minimum(base + rowc[g], rows_w - 1)
                if0[j, sl] = plsc.load_gather(ir0, [row, colc[g]])
                if1[j, sl] = plsc.load_gather(ir1, [row, colc[g]])
            return carry

        lax.fori_loop(0, chunks, flatten, 0)

        def issue_gathers(j, r0x, r1x, sgx):
            pltpu.async_copy(ct_hbm.at[if0.at[j, pl.ds(0, lpc)]], r0x, sgx)
            pltpu.async_copy(ct_hbm.at[if1.at[j, pl.ds(0, lpc)]], r1x, sgx)

        def wait_gathers(r0x, r1x, sgx):
            pltpu.make_async_copy(
                ct_hbm.at[if0.at[0, pl.ds(0, lpc)]], r0x, sgx).wait()
            pltpu.make_async_copy(
                ct_hbm.at[if1.at[0, pl.ds(0, lpc)]], r1x, sgx).wait()

        def out_dst(j):
            return out_hbm.at[pl.ds(row0 + j * RPC, RPC),
                              pl.ds(0, fields), pl.ds(0, DIM)]

        def compute(r0x, r1x, ox):
            @plsc.parallel_loop(0, lpc, 1, unroll=8)
            def row(k):
                i = k // fields
                j = k - i * fields
                a = [r0x[k, pl.ds(16 * q, 16)]
                     + r1x[k, pl.ds(DIM + 16 * q, 16)]
                     for q in range(NQ)]
                tot = jnp.sum((a[0] + a[1]) + (a[2] + a[3]))
                tot2 = jnp.sum((a[0] * a[0] + a[1] * a[1])
                               + (a[2] * a[2] + a[3] * a[3]))
                mu = tot * (1.0 / DIM)
                var = tot2 * (1.0 / DIM) - mu * mu
                rstd = _rsqrt_newton(var + EPS)
                for q in range(NQ):
                    ox[i, j, pl.ds(16 * q, 16)] = \
                        (a[q] - mu) * (rstd * gv[q]) + bv[q]

        issue_gathers(0, r0a, r1a, sga)
        pltpu.async_copy(oa, out_dst(0), soa)
        pltpu.async_copy(ob, out_dst(1), sob)

        def pair(p, carry):
            ja = 2 * p
            wait_gathers(r0a, r1a, sga)
            issue_gathers(ja + 1, r0b, r1b, sgb)
            pltpu.make_async_copy(oa, out_dst(0), soa).wait()
            compute(r0a, r1a, oa)
            pltpu.async_copy(oa, out_dst(ja), soa)
            wait_gathers(r0b, r1b, sgb)

            @pl.when(p < chunks // 2 - 1)
            def _():
                issue_gathers(ja + 2, r0a, r1a, sga)

            pltpu.make_async_copy(ob, out_dst(0), sob).wait()
            compute(r0b, r1b, ob)
            pltpu.async_copy(ob, out_dst(ja + 1), sob)
            return carry

        lax.fori_loop(0, chunks // 2, pair, 0)
        pltpu.make_async_copy(oa, out_dst(0), soa).wait()
        pltpu.make_async_copy(ob, out_dst(0), sob).wait()

    f = pl.kernel(
        body,
        out_type=jax.ShapeDtypeStruct((batch, PADF, LANES), jnp.float32),
        mesh=mesh,
        scratch_types=[
            pltpu.VMEM((rows_w, LANES), jnp.int32),
            pltpu.VMEM((rows_w, LANES), jnp.int32),
            pltpu.VMEM((chunks, LANES), jnp.int32),
            pltpu.VMEM((chunks, LANES), jnp.int32),
            pltpu.VMEM((2, DIM), jnp.float32),
            pltpu.VMEM((lpc, 2 * DIM), jnp.float32),
            pltpu.VMEM((lpc, 2 * DIM), jnp.float32),
            pltpu.VMEM((lpc, 2 * DIM), jnp.float32),
            pltpu.VMEM((lpc, 2 * DIM), jnp.float32),
            pltpu.VMEM((RPC, fields, DIM), jnp.float32),
            pltpu.VMEM((RPC, fields, DIM), jnp.float32),
            pltpu.SemaphoreType.DMA,
            pltpu.SemaphoreType.DMA,
            pltpu.SemaphoreType.DMA,
            pltpu.SemaphoreType.DMA,
        ],
        compiler_params=pltpu.CompilerParams(
            use_tc_tiling_on_sc=False, needs_layout_passes=False),
    )
    return f(i0, i1, ct, gamma, beta)


def kernel(idx0, idx1, table0, table1, gamma, beta):
    pad = ((0, 0), (0, 128 - idx0.shape[1]))
    i0 = jnp.pad(idx0.astype(jnp.int32), pad)
    i1 = jnp.pad(idx1.astype(jnp.int32), pad)
    ct = _concat_tables(table0, table1)
    padded = _fused_sc(i0, i1, ct, gamma, beta, idx0.shape[1])
    return padded[:, :idx0.shape[1], :DIM]


# speedup vs baseline: 1.4515x; 1.4515x over previous
import jax
import jax.numpy as jnp
from jax import lax
from jax.experimental import pallas as pl
from jax.experimental.pallas import tpu as pltpu
from jax.experimental.pallas import tpu_sc as plsc

DIM = 64
EPS = 1e-5
NC, NS = 2, 16
NW = NC * NS
RPC = 4
NQ = DIM // 16
LANES = 128
PADF = 32


def _rsqrt_newton(x):
    i = lax.bitcast_convert_type(x, jnp.int32)
    i = jnp.int32(0x5F3759DF) - (i >> 1)
    y = lax.bitcast_convert_type(i, jnp.float32)
    for _ in range(3):
        y = y * (1.5 - 0.5 * x * y * y)
    return y


def _build_ct(tt0, tt1, vpad):
    blk = 4096
    assert vpad % blk == 0

    def body(a_ref, b_ref, o_ref):
        o_ref[...] = jnp.concatenate(
            [a_ref[...].T, b_ref[...].T], axis=-1)

    return pl.pallas_call(
        body,
        grid=(vpad // blk,),
        in_specs=[
            pl.BlockSpec((DIM, blk), lambda i: (0, i)),
            pl.BlockSpec((DIM, blk), lambda i: (0, i)),
        ],
        out_specs=pl.BlockSpec((blk, 2 * DIM), lambda i: (i, 0)),
        out_shape=jax.ShapeDtypeStruct((vpad, 2 * DIM), jnp.float32),
    )(tt0, tt1)


def _fused_sc(i0, i1, ct, gamma, beta, fields):
    batch = i0.shape[0]
    rows_w = batch // NW
    chunks = rows_w // RPC
    lpc = RPC * fields
    assert chunks % 2 == 0 and lpc % 8 == 0 and lpc <= LANES
    groups = (lpc + 15) // 16
    mesh = plsc.VectorSubcoreMesh(core_axis_name="c", subcore_axis_name="s")

    def body(i0_hbm, i1_hbm, ct_hbm, g_hbm, b_hbm, out_hbm,
             ir0, ir1, if0, if1, gb_v,
             r0a, r1a, r0b, r1b, oa, ob,
             sga, sgb, soa, sob):
        wid = lax.axis_index("s") * NC + lax.axis_index("c")
        row0 = wid * rows_w

        pltpu.sync_copy(i0_hbm.at[pl.ds(row0, rows_w)], ir0)
        pltpu.sync_copy(i1_hbm.at[pl.ds(row0, rows_w)], ir1)
        pltpu.sync_copy(g_hbm, gb_v.at[0])
        pltpu.sync_copy(b_hbm, gb_v.at[1])
        gv = [gb_v[0, pl.ds(16 * q, 16)] for q in range(NQ)]
        bv = [gb_v[1, pl.ds(16 * q, 16)] for q in range(NQ)]

        rowc = []
        colc = []
        for g in range(groups):
            t = jnp.arange(16, dtype=jnp.int32) + 16 * g
            rowc.append(t // fields)
            colc.append(t % fields)

        def flatten(j, carry):
            base = j * RPC
            for g in range(groups):
                sl = pl.ds(16 * g, 16)
                row = jnp.minimum(base + rowc[g], rows_w - 1)
                if0[j, sl] = plsc.load_gather(ir0, [row, colc[g]])
                if1[j, sl] = plsc.load_gather(ir1, [row, colc[g]])
            return carry

        lax.fori_loop(0, chunks, flatten, 0)

        def issue_gathers(j, r0x, r1x, sgx):
            pltpu.async_copy(ct_hbm.at[if0.at[j, pl.ds(0, lpc)]], r0x, sgx)
            pltpu.async_copy(ct_hbm.at[if1.at[j, pl.ds(0, lpc)]], r1x, sgx)

        def wait_gathers(r0x, r1x, sgx):
            pltpu.make_async_copy(
                ct_hbm.at[if0.at[0, pl.ds(0, lpc)]], r0x, sgx).wait()
            pltpu.make_async_copy(
                ct_hbm.at[if1.at[0, pl.ds(0, lpc)]], r1x, sgx).wait()

        def out_dst(j):
            return out_hbm.at[pl.ds(row0 + j * RPC, RPC),
                              pl.ds(0, fields), pl.ds(0, DIM)]

        def compute(r0x, r1x, ox):
            @plsc.parallel_loop(0, lpc, 1, unroll=8)
            def row(k):
                i = k // fields
                j = k - i * fields
                a = [r0x[k, pl.ds(16 * q, 16)]
                     + r1x[k, pl.ds(DIM + 16 * q, 16)]
                     for q in range(NQ)]
                tot = jnp.sum((a[0] + a[1]) + (a[2] + a[3]))
                tot2 = jnp.sum((a[0] * a[0] + a[1] * a[1])
                               + (a[2] * a[2] + a[3] * a[3]))
                mu = tot * (1.0 / DIM)
                var = tot2 * (1.0 / DIM) - mu * mu
                rstd = _rsqrt_newton(var + EPS)
                for q in range(NQ):
                    ox[i, j, pl.ds(16 * q, 16)] = \
                        (a[q] - mu) * (rstd * gv[q]) + bv[q]

        issue_gathers(0, r0a, r1a, sga)
        pltpu.async_copy(oa, out_dst(0), soa)
        pltpu.async_copy(ob, out_dst(1), sob)

        def pair(p, carry):
            ja = 2 * p
            wait_gathers(r0a, r1a, sga)
            issue_gathers(ja + 1, r0b, r1b, sgb)
            pltpu.make_async_copy(oa, out_dst(0), soa).wait()
            compute(r0a, r1a, oa)
            pltpu.async_copy(oa, out_dst(ja), soa)
            wait_gathers(r0b, r1b, sgb)

            @pl.when(p < chunks // 2 - 1)
            def _():
                issue_gathers(ja + 2, r0a, r1a, sga)

            pltpu.make_async_copy(ob, out_dst(0), sob).wait()
            compute(r0b, r1b, ob)
            pltpu.async_copy(ob, out_dst(ja + 1), sob)
            return carry

        lax.fori_loop(0, chunks // 2, pair, 0)
        pltpu.make_async_copy(oa, out_dst(0), soa).wait()
        pltpu.make_async_copy(ob, out_dst(0), sob).wait()

    f = pl.kernel(
        body,
        out_type=jax.ShapeDtypeStruct((batch, PADF, LANES), jnp.float32),
        mesh=mesh,
        scratch_types=[
            pltpu.VMEM((rows_w, LANES), jnp.int32),
            pltpu.VMEM((rows_w, LANES), jnp.int32),
            pltpu.VMEM((chunks, LANES), jnp.int32),
            pltpu.VMEM((chunks, LANES), jnp.int32),
            pltpu.VMEM((2, DIM), jnp.float32),
            pltpu.VMEM((lpc, 2 * DIM), jnp.float32),
            pltpu.VMEM((lpc, 2 * DIM), jnp.float32),
            pltpu.VMEM((lpc, 2 * DIM), jnp.float32),
            pltpu.VMEM((lpc, 2 * DIM), jnp.float32),
            pltpu.VMEM((RPC, fields, DIM), jnp.float32),
            pltpu.VMEM((RPC, fields, DIM), jnp.float32),
            pltpu.SemaphoreType.DMA,
            pltpu.SemaphoreType.DMA,
            pltpu.SemaphoreType.DMA,
            pltpu.SemaphoreType.DMA,
        ],
        compiler_params=pltpu.CompilerParams(
            use_tc_tiling_on_sc=False, needs_layout_passes=False),
    )
    return f(i0, i1, ct, gamma, beta)


def kernel(idx0, idx1, table0, table1, gamma, beta):
    pad = ((0, 0), (0, 128 - idx0.shape[1]))
    i0 = jnp.pad(idx0.astype(jnp.int32), pad)
    i1 = jnp.pad(idx1.astype(jnp.int32), pad)
    vpad = ((table0.shape[0] + 4095) // 4096) * 4096
    ct = _build_ct(table0.T, table1.T, vpad)
    padded = _fused_sc(i0, i1, ct, gamma, beta, idx0.shape[1])
    return padded[:, :idx0.shape[1], :DIM]


# speedup vs baseline: 1.4892x; 1.0260x over previous
import jax
import jax.numpy as jnp
from jax import lax
from jax.experimental import pallas as pl
from jax.experimental.pallas import tpu as pltpu
from jax.experimental.pallas import tpu_sc as plsc

DIM = 64
EPS = 1e-5
NC, NS = 2, 16
NW = NC * NS
RPC = 4
NQ = DIM // 16
LANES = 128
PADF = 32


def _rsqrt_newton(x):
    i = lax.bitcast_convert_type(x, jnp.int32)
    i = jnp.int32(0x5F3759DF) - (i >> 1)
    y = lax.bitcast_convert_type(i, jnp.float32)
    for _ in range(3):
        y = y * (1.5 - 0.5 * x * y * y)
    return y


def _build_ct(tt0, tt1, vpad):
    blk = 8192
    assert vpad % blk == 0

    def body(a_ref, b_ref, o_ref):
        o_ref[...] = jnp.concatenate(
            [a_ref[...].T, b_ref[...].T], axis=-1)

    return pl.pallas_call(
        body,
        grid=(vpad // blk,),
        in_specs=[
            pl.BlockSpec((DIM, blk), lambda i: (0, i)),
            pl.BlockSpec((DIM, blk), lambda i: (0, i)),
        ],
        out_specs=pl.BlockSpec((blk, 2 * DIM), lambda i: (i, 0)),
        out_shape=jax.ShapeDtypeStruct((vpad, 2 * DIM), jnp.float32),
    )(tt0, tt1)


def _fused_sc(i0, i1, ct, gamma, beta, fields):
    batch = i0.shape[0]
    rows_w = batch // NW
    chunks = rows_w // RPC
    lpc = RPC * fields
    assert chunks % 2 == 0 and lpc % 8 == 0 and lpc <= LANES
    groups = (lpc + 15) // 16
    mesh = plsc.VectorSubcoreMesh(core_axis_name="c", subcore_axis_name="s")

    def body(i0_hbm, i1_hbm, ct_hbm, g_hbm, b_hbm, out_hbm,
             ir0, ir1, if0, if1, gb_v,
             r0a, r1a, r0b, r1b, oa, ob,
             sga, sgb, soa, sob):
        wid = lax.axis_index("s") * NC + lax.axis_index("c")
        row0 = wid * rows_w

        pltpu.sync_copy(i0_hbm.at[pl.ds(row0, rows_w)], ir0)
        pltpu.sync_copy(i1_hbm.at[pl.ds(row0, rows_w)], ir1)
        pltpu.sync_copy(g_hbm, gb_v.at[0])
        pltpu.sync_copy(b_hbm, gb_v.at[1])
        gv = [gb_v[0, pl.ds(16 * q, 16)] for q in range(NQ)]
        bv = [gb_v[1, pl.ds(16 * q, 16)] for q in range(NQ)]

        rowc = []
        colc = []
        for g in range(groups):
            t = jnp.arange(16, dtype=jnp.int32) + 16 * g
            rowc.append(t // fields)
            colc.append(t % fields)

        def flatten(j, carry):
            base = j * RPC
            for g in range(groups):
                sl = pl.ds(16 * g, 16)
                row = jnp.minimum(base + rowc[g], rows_w - 1)
                if0[j, sl] = plsc.load_gather(ir0, [row, colc[g]])
                if1[j, sl] = plsc.load_gather(ir1, [row, colc[g]])
            return carry

        lax.fori_loop(0, chunks, flatten, 0)

        def issue_gathers(j, r0x, r1x, sgx):
            pltpu.async_copy(ct_hbm.at[if0.at[j, pl.ds(0, lpc)]], r0x, sgx)
            pltpu.async_copy(ct_hbm.at[if1.at[j, pl.ds(0, lpc)]], r1x, sgx)

        def wait_gathers(r0x, r1x, sgx):
            pltpu.make_async_copy(
                ct_hbm.at[if0.at[0, pl.ds(0, lpc)]], r0x, sgx).wait()
            pltpu.make_async_copy(
                ct_hbm.at[if1.at[0, pl.ds(0, lpc)]], r1x, sgx).wait()

        def out_dst(j):
            return out_hbm.at[pl.ds(row0 + j * RPC, RPC),
                              pl.ds(0, fields), pl.ds(0, DIM)]

        def compute(r0x, r1x, ox):
            @plsc.parallel_loop(0, lpc, 1, unroll=8)
            def row(k):
                i = k // fields
                j = k - i * fields
                a = [r0x[k, pl.ds(16 * q, 16)]
                     + r1x[k, pl.ds(DIM + 16 * q, 16)]
                     for q in range(NQ)]
                tot = jnp.sum((a[0] + a[1]) + (a[2] + a[3]))
                tot2 = jnp.sum((a[0] * a[0] + a[1] * a[1])
                               + (a[2] * a[2] + a[3] * a[3]))
                mu = tot * (1.0 / DIM)
                var = tot2 * (1.0 / DIM) - mu * mu
                rstd = _rsqrt_newton(var + EPS)
                for q in range(NQ):
                    ox[i, j, pl.ds(16 * q, 16)] = \
                        (a[q] - mu) * (rstd * gv[q]) + bv[q]

        issue_gathers(0, r0a, r1a, sga)
        pltpu.async_copy(oa, out_dst(0), soa)
        pltpu.async_copy(ob, out_dst(1), sob)

        def pair(p, carry):
            ja = 2 * p
            wait_gathers(r0a, r1a, sga)
            issue_gathers(ja + 1, r0b, r1b, sgb)
            pltpu.make_async_copy(oa, out_dst(0), soa).wait()
            compute(r0a, r1a, oa)
            pltpu.async_copy(oa, out_dst(ja), soa)
            wait_gathers(r0b, r1b, sgb)

            @pl.when(p < chunks // 2 - 1)
            def _():
                issue_gathers(ja + 2, r0a, r1a, sga)

            pltpu.make_async_copy(ob, out_dst(0), sob).wait()
            compute(r0b, r1b, ob)
            pltpu.async_copy(ob, out_dst(ja + 1), sob)
            return carry

        lax.fori_loop(0, chunks // 2, pair, 0)
        pltpu.make_async_copy(oa, out_dst(0), soa).wait()
        pltpu.make_async_copy(ob, out_dst(0), sob).wait()

    f = pl.kernel(
        body,
        out_type=jax.ShapeDtypeStruct((batch, PADF, LANES), jnp.float32),
        mesh=mesh,
        scratch_types=[
            pltpu.VMEM((rows_w, LANES), jnp.int32),
            pltpu.VMEM((rows_w, LANES), jnp.int32),
            pltpu.VMEM((chunks, LANES), jnp.int32),
            pltpu.VMEM((chunks, LANES), jnp.int32),
            pltpu.VMEM((2, DIM), jnp.float32),
            pltpu.VMEM((lpc, 2 * DIM), jnp.float32),
            pltpu.VMEM((lpc, 2 * DIM), jnp.float32),
            pltpu.VMEM((lpc, 2 * DIM), jnp.float32),
            pltpu.VMEM((lpc, 2 * DIM), jnp.float32),
            pltpu.VMEM((RPC, fields, DIM), jnp.float32),
            pltpu.VMEM((RPC, fields, DIM), jnp.float32),
            pltpu.SemaphoreType.DMA,
            pltpu.SemaphoreType.DMA,
            pltpu.SemaphoreType.DMA,
            pltpu.SemaphoreType.DMA,
        ],
        compiler_params=pltpu.CompilerParams(
            use_tc_tiling_on_sc=False, needs_layout_passes=False),
    )
    return f(i0, i1, ct, gamma, beta)


def kernel(idx0, idx1, table0, table1, gamma, beta):
    pad = ((0, 0), (0, 128 - idx0.shape[1]))
    i0 = jnp.pad(idx0.astype(jnp.int32), pad)
    i1 = jnp.pad(idx1.astype(jnp.int32), pad)
    vpad = ((table0.shape[0] + 8191) // 8192) * 8192
    ct = _build_ct(table0.T, table1.T, vpad)
    padded = _fused_sc(i0, i1, ct, gamma, beta, idx0.shape[1])
    return padded[:, :idx0.shape[1], :DIM]
